# baseline (speedup 1.0000x reference)
import functools

import jax
import jax.numpy as jnp
from jax.experimental import pallas as pl
from jax.experimental.pallas import tpu as pltpu

H = 224
W = 224
C = 384
NPTS = 2


def _floor_f32(v):
    t = v.astype(jnp.int32)
    return jnp.where(t.astype(jnp.float32) > v, t - 1, t)


def _body(grid_ref, x_ref, out_ref, row00, row01, row10, row11, sem):
    rows = (row00, row01, row10, row11)

    copies = []
    pix = []
    for p in range(NPTS):
        gx = grid_ref[p, 0]
        gy = grid_ref[p, 1]
        ix = ((gx + 1.0) * W - 1.0) / 2.0
        iy = ((gy + 1.0) * H - 1.0) / 2.0
        ix = jnp.clip(ix, -2.0, float(W) + 1.0)
        iy = jnp.clip(iy, -2.0, float(H) + 1.0)
        x0 = _floor_f32(ix)
        y0 = _floor_f32(iy)
        fx = ix - x0.astype(jnp.float32)
        fy = iy - y0.astype(jnp.float32)
        xs = []
        for cx in range(2):
            xc = x0 + cx
            wxc = fx if cx == 1 else 1.0 - fx
            vx = ((xc >= 0) & (xc <= W - 1)).astype(jnp.float32)
            xs.append((jnp.clip(xc, 0, W - 1), wxc * vx))
        pix.append(xs)
        for cy in range(2):
            yc = y0 + cy
            wyc = fy if cy == 1 else 1.0 - fy
            vy = ((yc >= 0) & (yc <= H - 1)).astype(jnp.float32)
            yi = jnp.clip(yc, 0, H - 1)
            dst = rows[p * 2 + cy]
            cp = pltpu.make_async_copy(x_ref.at[0, :, yi, :], dst, sem)
            cp.start()
            copies.append((cp, wyc * vy))

    lanes = jax.lax.broadcasted_iota(jnp.int32, (1, W), 1)

    for p in range(NPTS):
        acc = jnp.zeros((C, 1), jnp.float32)
        for cy in range(2):
            cp, wy = copies[p * 2 + cy]
            cp.wait()
            band = rows[p * 2 + cy][...]
            for cx in range(2):
                xi, wx = pix[p][cx]
                col = jnp.where(lanes == xi, band, 0.0).sum(
                    axis=1, keepdims=True)
                acc = acc + col * (wx * wy)
        out_ref[:, pl.ds(p, 1)] = acc


_call = pl.pallas_call(
    _body,
    out_shape=jax.ShapeDtypeStruct((C, NPTS), jnp.float32),
    in_specs=[
        pl.BlockSpec(memory_space=pltpu.MemorySpace.SMEM),
        pl.BlockSpec(memory_space=pltpu.MemorySpace.HBM),
    ],
    out_specs=pl.BlockSpec(memory_space=pltpu.MemorySpace.VMEM),
    scratch_shapes=[
        pltpu.VMEM((C, W), jnp.float32),
        pltpu.VMEM((C, W), jnp.float32),
        pltpu.VMEM((C, W), jnp.float32),
        pltpu.VMEM((C, W), jnp.float32),
        pltpu.SemaphoreType.DMA,
    ],
)


@jax.jit
def kernel(x, grid):
    out = _call(grid.reshape(NPTS, 2), x)
    return out.reshape(1, C, 1, NPTS)


# speedup vs baseline: 36.1782x; 36.1782x over previous
import jax
import jax.numpy as jnp
from jax.experimental import pallas as pl
from jax.experimental.pallas import tpu as pltpu

H = 224
W = 224
C = 384
NPTS = 2


def _floor_i32(v):
    t = v.astype(jnp.int32)
    return jnp.where(t.astype(jnp.float32) > v, t - 1, t)


def _corner_setup(g, size):
    i = ((g + 1.0) * size - 1.0) / 2.0
    i = jnp.clip(i, -2.0, float(size) + 1.0)
    c0 = _floor_i32(i)
    f = i - c0.astype(jnp.float32)
    base = jnp.clip(c0, 0, size - 2)
    out = []
    for corner in range(2):
        cc = c0 + corner
        wc = f if corner == 1 else 1.0 - f
        v = ((cc >= 0) & (cc <= size - 1)).astype(jnp.float32)
        out.append((jnp.clip(cc, 0, size - 1) - base, wc * v))
    (o0, w0), (o1, w1) = out
    return base, o0, o1, w0, w1


def _body(grid_ref, x_ref, out_ref, patch0, patch1, sem0, sem1):
    patches = (patch0, patch1)
    sems = (sem0, sem1)

    setups = []
    copies = []
    for p in range(NPTS):
        xs = _corner_setup(grid_ref[p, 0], W)
        ys = _corner_setup(grid_ref[p, 1], H)
        wb = pl.multiple_of(jnp.minimum((xs[0] >> 3) << 3, W - 16), 8)
        xs = (wb, xs[0] + xs[1] - wb, xs[0] + xs[2] - wb, xs[3], xs[4])
        setups.append((xs, ys))
        cp = pltpu.make_async_copy(
            x_ref.at[pl.ds(ys[0], 2), pl.ds(wb, 16), :],
            patches[p], sems[p])
        cp.start()
        copies.append(cp)

    ia = jax.lax.broadcasted_iota(jnp.int32, (2, 16, 1), 0)
    ib = jax.lax.broadcasted_iota(jnp.int32, (2, 16, 1), 1)

    for p in range(NPTS):
        (_, ox0, ox1, wx0, wx1), (_, oy0, oy1, wy0, wy1) = setups[p]
        wgt = jnp.zeros((2, 16, 1), jnp.float32)
        for oy, wy in ((oy0, wy0), (oy1, wy1)):
            for ox, wx in ((ox0, wx0), (ox1, wx1)):
                wgt = wgt + jnp.where((ia == oy) & (ib == ox), wy * wx, 0.0)
        copies[p].wait()
        vals = patches[p][...] * wgt
        out_ref[pl.ds(p, 1), :] = vals.sum(axis=0).sum(axis=0, keepdims=True)


_call = pl.pallas_call(
    _body,
    out_shape=jax.ShapeDtypeStruct((NPTS, C), jnp.float32),
    in_specs=[
        pl.BlockSpec(memory_space=pltpu.MemorySpace.SMEM),
        pl.BlockSpec(memory_space=pltpu.MemorySpace.HBM),
    ],
    out_specs=pl.BlockSpec(memory_space=pltpu.MemorySpace.VMEM),
    scratch_shapes=[
        pltpu.VMEM((2, 16, C), jnp.float32),
        pltpu.VMEM((2, 16, C), jnp.float32),
        pltpu.SemaphoreType.DMA,
        pltpu.SemaphoreType.DMA,
    ],
)


@jax.jit
def kernel(x, grid):
    x_nhwc = jnp.transpose(x, (0, 2, 3, 1)).reshape(H, W, C)
    out = _call(grid.reshape(NPTS, 2), x_nhwc)
    return jnp.transpose(out).reshape(1, C, 1, NPTS)
